# baseline (speedup 1.0000x reference)
import jax
import jax.numpy as jnp
from jax import lax
from jax.experimental import pallas as pl
from jax.experimental.pallas import tpu as pltpu
from jax.experimental.pallas import tpu_sc as plsc

N = 10000
NP = 10016
E = 320000
C = 16
G = 128
NCORES = 2
NSUB = 16
NTILES = NCORES * NSUB
CHUNK = 128
CH_TOT = E // CHUNK
NCH0 = 79
NCH1 = 79
NCHMAX = max(NCH0, NCH1)
C1BASE = NSUB * NCH0
T31BASE = C1BASE + 15 * NCH1
BT31 = CH_TOT - T31BASE
PADCH = T31BASE + NCH1 - CH_TOT
GSZ = 16
RPT = NP // NSUB

_MESH = plsc.VectorSubcoreMesh(
    core_axis_name="c", subcore_axis_name="s",
    num_cores=NCORES, num_subcores=NSUB)


def _stage_indices(ei3, pad3, which, dst, cid, sid):
    @pl.when(cid == 0)
    def _():
        pltpu.sync_copy(ei3.at[which, pl.ds(sid * NCH0, NCH0)],
                        dst.at[pl.ds(0, NCH0)])

    @pl.when((cid == 1) & (sid < NSUB - 1))
    def _():
        pltpu.sync_copy(ei3.at[which, pl.ds(C1BASE + sid * NCH1, NCH1)],
                        dst.at[pl.ds(0, NCH1)])

    @pl.when((cid == 1) & (sid == NSUB - 1))
    def _():
        pltpu.sync_copy(ei3.at[which, pl.ds(T31BASE, BT31)],
                        dst.at[pl.ds(0, BT31)])
        pltpu.sync_copy(pad3.at[which], dst.at[pl.ds(BT31, PADCH)])


def _make_sc_body(with_gather):
    def body(*refs):
        if with_gather:
            (u_hbm, ei3, pad3, out_hbm,
             row_v, col_v, buf, zbuf, acc, u_sh, gsem, ssem) = refs
        else:
            ei3, pad3, out_hbm, col_v, buf, zbuf, acc, ssem = refs
        cid = lax.axis_index("c")
        sid = lax.axis_index("s")
        zeros16 = jnp.zeros((16,), jnp.float32)

        def zero_body(i, carry):
            zbuf[i, :] = zeros16
            return carry
        lax.fori_loop(0, RPT, zero_body, 0)
        pltpu.sync_copy(zbuf, acc.at[pl.ds(sid * RPT, RPT), :])
        if with_gather:
            pltpu.sync_copy(u_hbm.at[pl.ds(sid * RPT, RPT), :],
                            u_sh.at[pl.ds(sid * RPT, RPT), :])
            _stage_indices(ei3, pad3, 0, row_v, cid, sid)
        else:
            ones16 = jnp.ones((16,), jnp.float32)

            def ones_body(i, carry):
                buf[i, :] = ones16
                return carry
            lax.fori_loop(0, CHUNK, ones_body, 0)
        _stage_indices(ei3, pad3, 1, col_v, cid, sid)
        plsc.subcore_barrier()

        if with_gather:
            def issue_g(g, par, size):
                for b in range(size):
                    pltpu.async_copy(u_sh.at[row_v.at[g * GSZ + b]],
                                     buf.at[par, b], gsem)

            def issue_s(g, par, size):
                for b in range(size):
                    pltpu.async_copy(buf.at[par, b],
                                     acc.at[col_v.at[g * GSZ + b]],
                                     ssem, add=True)

            def drain(sem, k):
                for _ in range(k):
                    pltpu.make_async_copy(u_hbm.at[pl.ds(0, CHUNK), :],
                                          buf.at[0, 0], sem).wait()

            def pipeline(nch):
                ngf, tail = nch // GSZ, nch % GSZ
                tail_par = ngf % 2
                issue_g(0, 0, GSZ)

                def g_body(g, carry):
                    par = lax.rem(g, 2)
                    drain(gsem, GSZ)
                    issue_s(g, par, GSZ)

                    @pl.when(g >= 1)
                    def _():
                        drain(ssem, GSZ)

                    @pl.when(g + 1 < ngf)
                    def _():
                        issue_g(g + 1, 1 - par, GSZ)
                    return carry
                lax.fori_loop(0, ngf, g_body, 0)
                if tail:
                    issue_g(ngf, tail_par, tail)
                    drain(ssem, GSZ)
                    drain(gsem, tail)
                    issue_s(ngf, tail_par, tail)
                    drain(ssem, tail)
                else:
                    drain(ssem, GSZ)

            @pl.when(cid == 0)
            def _():
                pipeline(NCH0)

            @pl.when(cid == 1)
            def _():
                pipeline(NCH1)
        else:
            nch_t = jnp.where(cid == 0, NCH0, NCH1)

            def chunk_body(j, carry):
                pltpu.async_copy(buf, acc.at[col_v.at[j]], ssem, add=True)
                return carry
            lax.fori_loop(0, nch_t, chunk_body, 0)

            def drain_body(j, carry):
                pltpu.make_async_copy(
                    buf, acc.at[pl.ds(0, CHUNK), :], ssem).wait()
                return carry
            lax.fori_loop(0, nch_t, drain_body, 0)
        plsc.subcore_barrier()
        pltpu.sync_copy(acc.at[pl.ds(sid * RPT, RPT), :],
                        out_hbm.at[cid, pl.ds(sid * RPT, RPT), :])
    return body


_round_kernel = pl.kernel(
    _make_sc_body(True),
    out_type=jax.ShapeDtypeStruct((NCORES, NP, C), jnp.float32),
    mesh=_MESH,
    compiler_params=pltpu.CompilerParams(use_tc_tiling_on_sc=False),
    scratch_types=[
        pltpu.VMEM((NCHMAX, CHUNK), jnp.int32),
        pltpu.VMEM((NCHMAX, CHUNK), jnp.int32),
        pltpu.VMEM((2, GSZ, CHUNK, C), jnp.float32),
        pltpu.VMEM((RPT, C), jnp.float32),
        pltpu.VMEM_SHARED((NP, C), jnp.float32),
        pltpu.VMEM_SHARED((NP, C), jnp.float32),
        pltpu.SemaphoreType.DMA,
        pltpu.SemaphoreType.DMA,
    ],
)

_deg_kernel = pl.kernel(
    _make_sc_body(False),
    out_type=jax.ShapeDtypeStruct((NCORES, NP, C), jnp.float32),
    mesh=_MESH,
    compiler_params=pltpu.CompilerParams(use_tc_tiling_on_sc=False),
    scratch_types=[
        pltpu.VMEM((NCHMAX, CHUNK), jnp.int32),
        pltpu.VMEM((CHUNK, C), jnp.float32),
        pltpu.VMEM((RPT, C), jnp.float32),
        pltpu.VMEM_SHARED((NP, C), jnp.float32),
        pltpu.SemaphoreType.DMA,
    ],
)


PK = NP * C // 128
PKN = N * C // 128


def _prep_body(x8_ref, w_ref, degp_ref, u0_ref, dis_ref):
    dis = lax.rsqrt(degp_ref[0] + degp_ref[1] + 1.0)
    w = w_ref[...]
    blocks = []
    for j in range(8):
        parts = []
        if j:
            parts.append(jnp.zeros((128, C * j), jnp.float32))
        parts.append(w)
        if j < 7:
            parts.append(jnp.zeros((128, C * (7 - j)), jnp.float32))
        blocks.append(jnp.concatenate(parts, axis=1) if len(parts) > 1
                      else parts[0])
    wblk = jnp.concatenate(blocks, axis=0)
    ypk = jnp.dot(x8_ref[...], wblk, preferred_element_type=jnp.float32)
    u0_ref[pl.ds(0, PKN), :] = dis[:PKN, :] * ypk
    u0_ref[pl.ds(PKN, PK - PKN), :] = jnp.zeros((PK - PKN, 128), jnp.float32)
    dis_ref[...] = dis


def _prep(x8, W, degp_pk):
    return pl.pallas_call(
        _prep_body,
        out_shape=(jax.ShapeDtypeStruct((PK, 128), jnp.float32),
                   jax.ShapeDtypeStruct((PK, 128), jnp.float32)),
    )(x8, W, degp_pk)


def _mid_body(p_ref, u_ref, dis_ref, out_ref):
    d = dis_ref[...]
    out_ref[...] = d * d * (p_ref[0] + p_ref[1] + u_ref[...])


def _mid(p_pk, u_pk, dis_pk):
    return pl.pallas_call(
        _mid_body,
        out_shape=jax.ShapeDtypeStruct((PK, 128), jnp.float32),
    )(p_pk, u_pk, dis_pk)


def _final_body(p_ref, u_ref, dis_ref, batchj_ref, b_ref, out_ref):
    d = dis_ref[...]
    h2 = d * (p_ref[0] + p_ref[1] + u_ref[...])
    h2n = h2[:PKN, :]
    gids = lax.broadcasted_iota(jnp.int32, (G, PKN), 0)
    sums = jnp.zeros((G, C), jnp.float32)
    cnt = jnp.zeros((G, 1), jnp.float32)
    for j in range(8):
        oh = (gids == batchj_ref[j:j + 1, :]).astype(jnp.float32)
        sj = jnp.dot(oh, h2n, preferred_element_type=jnp.float32)
        sums = sums + sj[:, C * j:C * (j + 1)]
        cnt = cnt + jnp.sum(oh, axis=1, keepdims=True)
    mean = sums / jnp.maximum(cnt, 1.0) + b_ref[...] * jnp.minimum(cnt, 1.0)
    m = jnp.max(mean, axis=1, keepdims=True)
    lse = jnp.log(jnp.sum(jnp.exp(mean - m), axis=1, keepdims=True)) + m
    out_ref[...] = mean - lse


def _final(p_pk, u_pk, dis_pk, batchj, b2):
    return pl.pallas_call(
        _final_body,
        out_shape=jax.ShapeDtypeStruct((G, C), jnp.float32),
    )(p_pk, u_pk, dis_pk, batchj, b2)


def kernel(x, edge_index, batch, W, b):
    ei3 = edge_index.reshape(2, CH_TOT, CHUNK)
    pad3 = jnp.full((2, PADCH, CHUNK), N, jnp.int32)
    x8 = x.reshape(PKN, 1024)
    batchj = batch.reshape(PKN, 8).T
    b2 = b.reshape(1, C)

    degp = _deg_kernel(ei3, pad3)
    u0_pk, dis_pk = _prep(x8, W, degp.reshape(2, PK, 128))
    pA = _round_kernel(u0_pk.reshape(NP, C), ei3, pad3)
    u1_pk = _mid(pA.reshape(2, PK, 128), u0_pk, dis_pk)
    pB = _round_kernel(u1_pk.reshape(NP, C), ei3, pad3)
    return _final(pB.reshape(2, PK, 128), u1_pk, dis_pk, batchj, b2)


# speedup vs baseline: 1.0352x; 1.0352x over previous
import jax
import jax.numpy as jnp
from jax import lax
from jax.experimental import pallas as pl
from jax.experimental.pallas import tpu as pltpu
from jax.experimental.pallas import tpu_sc as plsc

N = 10000
NP = 10016
E = 320000
C = 16
G = 128
NCORES = 2
NSUB = 16
NTILES = NCORES * NSUB
CHUNK = 128
CH_TOT = E // CHUNK
NCH0 = 84
NCH1 = 74
NCHMAX = max(NCH0, NCH1)
C1BASE = NSUB * NCH0
T31BASE = C1BASE + 15 * NCH1
BT31 = CH_TOT - T31BASE
PADCH = T31BASE + NCH1 - CH_TOT
GSZ = 16
RPT = NP // NSUB

_MESH = plsc.VectorSubcoreMesh(
    core_axis_name="c", subcore_axis_name="s",
    num_cores=NCORES, num_subcores=NSUB)


def _stage_indices(ei3, pad3, which, dst, cid, sid):
    @pl.when(cid == 0)
    def _():
        pltpu.sync_copy(ei3.at[which, pl.ds(sid * NCH0, NCH0)],
                        dst.at[pl.ds(0, NCH0)])

    @pl.when((cid == 1) & (sid < NSUB - 1))
    def _():
        pltpu.sync_copy(ei3.at[which, pl.ds(C1BASE + sid * NCH1, NCH1)],
                        dst.at[pl.ds(0, NCH1)])

    @pl.when((cid == 1) & (sid == NSUB - 1))
    def _():
        pltpu.sync_copy(ei3.at[which, pl.ds(T31BASE, BT31)],
                        dst.at[pl.ds(0, BT31)])
        pltpu.sync_copy(pad3.at[which], dst.at[pl.ds(BT31, PADCH)])


def _make_sc_body(with_gather):
    def body(*refs):
        if with_gather:
            (u_hbm, ei3, pad3, out_hbm,
             row_v, col_v, buf, zbuf, acc, u_sh, gsem, ssem) = refs
        else:
            ei3, pad3, out_hbm, col_v, buf, zbuf, acc, ssem = refs
        cid = lax.axis_index("c")
        sid = lax.axis_index("s")
        zeros16 = jnp.zeros((16,), jnp.float32)

        def zero_body(i, carry):
            zbuf[i, :] = zeros16
            return carry
        lax.fori_loop(0, RPT, zero_body, 0)
        pltpu.sync_copy(zbuf, acc.at[pl.ds(sid * RPT, RPT), :])
        if with_gather:
            pltpu.sync_copy(u_hbm.at[pl.ds(sid * RPT, RPT), :],
                            u_sh.at[pl.ds(sid * RPT, RPT), :])
            _stage_indices(ei3, pad3, 0, row_v, cid, sid)
        else:
            ones16 = jnp.ones((16,), jnp.float32)

            def ones_body(i, carry):
                buf[i, :] = ones16
                return carry
            lax.fori_loop(0, CHUNK, ones_body, 0)
        _stage_indices(ei3, pad3, 1, col_v, cid, sid)
        plsc.subcore_barrier()

        if with_gather:
            def issue_g(g, par, size):
                for b in range(size):
                    pltpu.async_copy(u_sh.at[row_v.at[g * GSZ + b]],
                                     buf.at[par, b], gsem)

            def issue_s(g, par, size):
                for b in range(size):
                    pltpu.async_copy(buf.at[par, b],
                                     acc.at[col_v.at[g * GSZ + b]],
                                     ssem, add=True)

            def drain(sem, k):
                for _ in range(k):
                    pltpu.make_async_copy(u_hbm.at[pl.ds(0, CHUNK), :],
                                          buf.at[0, 0], sem).wait()

            def pipeline(nch):
                ngf, tail = nch // GSZ, nch % GSZ
                tail_par = ngf % 2
                issue_g(0, 0, GSZ)

                def g_body(g, carry):
                    par = lax.rem(g, 2)
                    drain(gsem, GSZ)
                    issue_s(g, par, GSZ)

                    @pl.when(g >= 1)
                    def _():
                        drain(ssem, GSZ)

                    @pl.when(g + 1 < ngf)
                    def _():
                        issue_g(g + 1, 1 - par, GSZ)
                    return carry
                lax.fori_loop(0, ngf, g_body, 0)
                if tail:
                    issue_g(ngf, tail_par, tail)
                    drain(ssem, GSZ)
                    drain(gsem, tail)
                    issue_s(ngf, tail_par, tail)
                    drain(ssem, tail)
                else:
                    drain(ssem, GSZ)

            @pl.when(cid == 0)
            def _():
                pipeline(NCH0)

            @pl.when(cid == 1)
            def _():
                pipeline(NCH1)
        else:
            nch_t = jnp.where(cid == 0, NCH0, NCH1)

            def chunk_body(j, carry):
                pltpu.async_copy(buf, acc.at[col_v.at[j]], ssem, add=True)
                return carry
            lax.fori_loop(0, nch_t, chunk_body, 0)

            def drain_body(j, carry):
                pltpu.make_async_copy(
                    buf, acc.at[pl.ds(0, CHUNK), :], ssem).wait()
                return carry
            lax.fori_loop(0, nch_t, drain_body, 0)
        plsc.subcore_barrier()
        pltpu.sync_copy(acc.at[pl.ds(sid * RPT, RPT), :],
                        out_hbm.at[cid, pl.ds(sid * RPT, RPT), :])
    return body


_round_kernel = pl.kernel(
    _make_sc_body(True),
    out_type=jax.ShapeDtypeStruct((NCORES, NP, C), jnp.float32),
    mesh=_MESH,
    compiler_params=pltpu.CompilerParams(use_tc_tiling_on_sc=False),
    scratch_types=[
        pltpu.VMEM((NCHMAX, CHUNK), jnp.int32),
        pltpu.VMEM((NCHMAX, CHUNK), jnp.int32),
        pltpu.VMEM((2, GSZ, CHUNK, C), jnp.float32),
        pltpu.VMEM((RPT, C), jnp.float32),
        pltpu.VMEM_SHARED((NP, C), jnp.float32),
        pltpu.VMEM_SHARED((NP, C), jnp.float32),
        pltpu.SemaphoreType.DMA,
        pltpu.SemaphoreType.DMA,
    ],
)

_deg_kernel = pl.kernel(
    _make_sc_body(False),
    out_type=jax.ShapeDtypeStruct((NCORES, NP, C), jnp.float32),
    mesh=_MESH,
    compiler_params=pltpu.CompilerParams(use_tc_tiling_on_sc=False),
    scratch_types=[
        pltpu.VMEM((NCHMAX, CHUNK), jnp.int32),
        pltpu.VMEM((CHUNK, C), jnp.float32),
        pltpu.VMEM((RPT, C), jnp.float32),
        pltpu.VMEM_SHARED((NP, C), jnp.float32),
        pltpu.SemaphoreType.DMA,
    ],
)


PK = NP * C // 128
PKN = N * C // 128


def _prep_body(x8_ref, w_ref, degp_ref, u0_ref, dis_ref):
    dis = lax.rsqrt(degp_ref[0] + degp_ref[1] + 1.0)
    w = w_ref[...]
    blocks = []
    for j in range(8):
        parts = []
        if j:
            parts.append(jnp.zeros((128, C * j), jnp.float32))
        parts.append(w)
        if j < 7:
            parts.append(jnp.zeros((128, C * (7 - j)), jnp.float32))
        blocks.append(jnp.concatenate(parts, axis=1) if len(parts) > 1
                      else parts[0])
    wblk = jnp.concatenate(blocks, axis=0)
    ypk = jnp.dot(x8_ref[...], wblk, preferred_element_type=jnp.float32)
    u0_ref[pl.ds(0, PKN), :] = dis[:PKN, :] * ypk
    u0_ref[pl.ds(PKN, PK - PKN), :] = jnp.zeros((PK - PKN, 128), jnp.float32)
    dis_ref[...] = dis


def _prep(x8, W, degp_pk):
    return pl.pallas_call(
        _prep_body,
        out_shape=(jax.ShapeDtypeStruct((PK, 128), jnp.float32),
                   jax.ShapeDtypeStruct((PK, 128), jnp.float32)),
    )(x8, W, degp_pk)


def _mid_body(p_ref, u_ref, dis_ref, out_ref):
    d = dis_ref[...]
    out_ref[...] = d * d * (p_ref[0] + p_ref[1] + u_ref[...])


def _mid(p_pk, u_pk, dis_pk):
    return pl.pallas_call(
        _mid_body,
        out_shape=jax.ShapeDtypeStruct((PK, 128), jnp.float32),
    )(p_pk, u_pk, dis_pk)


def _final_body(p_ref, u_ref, dis_ref, batchj_ref, b_ref, out_ref):
    d = dis_ref[...]
    h2 = d * (p_ref[0] + p_ref[1] + u_ref[...])
    h2n = h2[:PKN, :]
    gids = lax.broadcasted_iota(jnp.int32, (G, PKN), 0)
    sums = jnp.zeros((G, C), jnp.float32)
    cnt = jnp.zeros((G, 1), jnp.float32)
    for j in range(8):
        oh = (gids == batchj_ref[j:j + 1, :]).astype(jnp.float32)
        sj = jnp.dot(oh, h2n, preferred_element_type=jnp.float32)
        sums = sums + sj[:, C * j:C * (j + 1)]
        cnt = cnt + jnp.sum(oh, axis=1, keepdims=True)
    mean = sums / jnp.maximum(cnt, 1.0) + b_ref[...] * jnp.minimum(cnt, 1.0)
    m = jnp.max(mean, axis=1, keepdims=True)
    lse = jnp.log(jnp.sum(jnp.exp(mean - m), axis=1, keepdims=True)) + m
    out_ref[...] = mean - lse


def _final(p_pk, u_pk, dis_pk, batchj, b2):
    return pl.pallas_call(
        _final_body,
        out_shape=jax.ShapeDtypeStruct((G, C), jnp.float32),
    )(p_pk, u_pk, dis_pk, batchj, b2)


def kernel(x, edge_index, batch, W, b):
    ei3 = edge_index.reshape(2, CH_TOT, CHUNK)
    pad3 = jnp.full((2, PADCH, CHUNK), N, jnp.int32)
    x8 = x.reshape(PKN, 1024)
    batchj = batch.reshape(PKN, 8).T
    b2 = b.reshape(1, C)

    degp = _deg_kernel(ei3, pad3)
    u0_pk, dis_pk = _prep(x8, W, degp.reshape(2, PK, 128))
    pA = _round_kernel(u0_pk.reshape(NP, C), ei3, pad3)
    u1_pk = _mid(pA.reshape(2, PK, 128), u0_pk, dis_pk)
    pB = _round_kernel(u1_pk.reshape(NP, C), ei3, pad3)
    return _final(pB.reshape(2, PK, 128), u1_pk, dis_pk, batchj, b2)


# speedup vs baseline: 1.0432x; 1.0078x over previous
import jax
import jax.numpy as jnp
from jax import lax
from jax.experimental import pallas as pl
from jax.experimental.pallas import tpu as pltpu
from jax.experimental.pallas import tpu_sc as plsc

N = 10000
NP = 10016
E = 320000
C = 16
G = 128
NCORES = 2
NSUB = 16
NTILES = NCORES * NSUB
CHUNK = 128
CH_TOT = E // CHUNK
NCH0 = 88
NCH1 = 70
NCHMAX = max(NCH0, NCH1)
C1BASE = NSUB * NCH0
T31BASE = C1BASE + 15 * NCH1
BT31 = CH_TOT - T31BASE
PADCH = T31BASE + NCH1 - CH_TOT
GSZ = 16
RPT = NP // NSUB

_MESH = plsc.VectorSubcoreMesh(
    core_axis_name="c", subcore_axis_name="s",
    num_cores=NCORES, num_subcores=NSUB)


def _stage_indices(ei3, pad3, which, dst, cid, sid):
    @pl.when(cid == 0)
    def _():
        pltpu.sync_copy(ei3.at[which, pl.ds(sid * NCH0, NCH0)],
                        dst.at[pl.ds(0, NCH0)])

    @pl.when((cid == 1) & (sid < NSUB - 1))
    def _():
        pltpu.sync_copy(ei3.at[which, pl.ds(C1BASE + sid * NCH1, NCH1)],
                        dst.at[pl.ds(0, NCH1)])

    @pl.when((cid == 1) & (sid == NSUB - 1))
    def _():
        pltpu.sync_copy(ei3.at[which, pl.ds(T31BASE, BT31)],
                        dst.at[pl.ds(0, BT31)])
        pltpu.sync_copy(pad3.at[which], dst.at[pl.ds(BT31, PADCH)])


def _make_sc_body(with_gather):
    def body(*refs):
        if with_gather:
            (u_hbm, ei3, pad3, out_hbm,
             row_v, col_v, buf, zbuf, acc, u_sh, gsem, ssem) = refs
        else:
            ei3, pad3, out_hbm, col_v, buf, zbuf, acc, ssem = refs
        cid = lax.axis_index("c")
        sid = lax.axis_index("s")
        zeros16 = jnp.zeros((16,), jnp.float32)

        def zero_body(i, carry):
            zbuf[i, :] = zeros16
            return carry
        lax.fori_loop(0, RPT, zero_body, 0)
        pltpu.sync_copy(zbuf, acc.at[pl.ds(sid * RPT, RPT), :])
        if with_gather:
            pltpu.sync_copy(u_hbm.at[pl.ds(sid * RPT, RPT), :],
                            u_sh.at[pl.ds(sid * RPT, RPT), :])
            _stage_indices(ei3, pad3, 0, row_v, cid, sid)
        else:
            ones16 = jnp.ones((16,), jnp.float32)

            def ones_body(i, carry):
                buf[i, :] = ones16
                return carry
            lax.fori_loop(0, CHUNK, ones_body, 0)
        _stage_indices(ei3, pad3, 1, col_v, cid, sid)
        plsc.subcore_barrier()

        if with_gather:
            def issue_g(g, par, size):
                for b in range(size):
                    pltpu.async_copy(u_sh.at[row_v.at[g * GSZ + b]],
                                     buf.at[par, b], gsem)

            def issue_s(g, par, size):
                for b in range(size):
                    pltpu.async_copy(buf.at[par, b],
                                     acc.at[col_v.at[g * GSZ + b]],
                                     ssem, add=True)

            def drain(sem, k):
                for _ in range(k):
                    pltpu.make_async_copy(u_hbm.at[pl.ds(0, CHUNK), :],
                                          buf.at[0, 0], sem).wait()

            def pipeline(nch):
                ngf, tail = nch // GSZ, nch % GSZ
                tail_par = ngf % 2
                issue_g(0, 0, GSZ)

                def g_body(g, carry):
                    par = lax.rem(g, 2)
                    drain(gsem, GSZ)
                    issue_s(g, par, GSZ)

                    @pl.when(g >= 1)
                    def _():
                        drain(ssem, GSZ)

                    @pl.when(g + 1 < ngf)
                    def _():
                        issue_g(g + 1, 1 - par, GSZ)
                    return carry
                lax.fori_loop(0, ngf, g_body, 0)
                if tail:
                    issue_g(ngf, tail_par, tail)
                    drain(ssem, GSZ)
                    drain(gsem, tail)
                    issue_s(ngf, tail_par, tail)
                    drain(ssem, tail)
                else:
                    drain(ssem, GSZ)

            @pl.when(cid == 0)
            def _():
                pipeline(NCH0)

            @pl.when(cid == 1)
            def _():
                pipeline(NCH1)
        else:
            nch_t = jnp.where(cid == 0, NCH0, NCH1)

            def chunk_body(j, carry):
                pltpu.async_copy(buf, acc.at[col_v.at[j]], ssem, add=True)
                return carry
            lax.fori_loop(0, nch_t, chunk_body, 0)

            def drain_body(j, carry):
                pltpu.make_async_copy(
                    buf, acc.at[pl.ds(0, CHUNK), :], ssem).wait()
                return carry
            lax.fori_loop(0, nch_t, drain_body, 0)
        plsc.subcore_barrier()
        pltpu.sync_copy(acc.at[pl.ds(sid * RPT, RPT), :],
                        out_hbm.at[cid, pl.ds(sid * RPT, RPT), :])
    return body


_round_kernel = pl.kernel(
    _make_sc_body(True),
    out_type=jax.ShapeDtypeStruct((NCORES, NP, C), jnp.float32),
    mesh=_MESH,
    compiler_params=pltpu.CompilerParams(use_tc_tiling_on_sc=False),
    scratch_types=[
        pltpu.VMEM((NCHMAX, CHUNK), jnp.int32),
        pltpu.VMEM((NCHMAX, CHUNK), jnp.int32),
        pltpu.VMEM((2, GSZ, CHUNK, C), jnp.float32),
        pltpu.VMEM((RPT, C), jnp.float32),
        pltpu.VMEM_SHARED((NP, C), jnp.float32),
        pltpu.VMEM_SHARED((NP, C), jnp.float32),
        pltpu.SemaphoreType.DMA,
        pltpu.SemaphoreType.DMA,
    ],
)

_deg_kernel = pl.kernel(
    _make_sc_body(False),
    out_type=jax.ShapeDtypeStruct((NCORES, NP, C), jnp.float32),
    mesh=_MESH,
    compiler_params=pltpu.CompilerParams(use_tc_tiling_on_sc=False),
    scratch_types=[
        pltpu.VMEM((NCHMAX, CHUNK), jnp.int32),
        pltpu.VMEM((CHUNK, C), jnp.float32),
        pltpu.VMEM((RPT, C), jnp.float32),
        pltpu.VMEM_SHARED((NP, C), jnp.float32),
        pltpu.SemaphoreType.DMA,
    ],
)


PK = NP * C // 128
PKN = N * C // 128


def _prep_body(x8_ref, w_ref, degp_ref, u0_ref, dis_ref):
    dis = lax.rsqrt(degp_ref[0] + degp_ref[1] + 1.0)
    w = w_ref[...]
    blocks = []
    for j in range(8):
        parts = []
        if j:
            parts.append(jnp.zeros((128, C * j), jnp.float32))
        parts.append(w)
        if j < 7:
            parts.append(jnp.zeros((128, C * (7 - j)), jnp.float32))
        blocks.append(jnp.concatenate(parts, axis=1) if len(parts) > 1
                      else parts[0])
    wblk = jnp.concatenate(blocks, axis=0)
    ypk = jnp.dot(x8_ref[...], wblk, preferred_element_type=jnp.float32)
    u0_ref[pl.ds(0, PKN), :] = dis[:PKN, :] * ypk
    u0_ref[pl.ds(PKN, PK - PKN), :] = jnp.zeros((PK - PKN, 128), jnp.float32)
    dis_ref[...] = dis


def _prep(x8, W, degp_pk):
    return pl.pallas_call(
        _prep_body,
        out_shape=(jax.ShapeDtypeStruct((PK, 128), jnp.float32),
                   jax.ShapeDtypeStruct((PK, 128), jnp.float32)),
    )(x8, W, degp_pk)


def _mid_body(p_ref, u_ref, dis_ref, out_ref):
    d = dis_ref[...]
    out_ref[...] = d * d * (p_ref[0] + p_ref[1] + u_ref[...])


def _mid(p_pk, u_pk, dis_pk):
    return pl.pallas_call(
        _mid_body,
        out_shape=jax.ShapeDtypeStruct((PK, 128), jnp.float32),
    )(p_pk, u_pk, dis_pk)


def _final_body(p_ref, u_ref, dis_ref, batchj_ref, b_ref, out_ref):
    d = dis_ref[...]
    h2 = d * (p_ref[0] + p_ref[1] + u_ref[...])
    h2n = h2[:PKN, :]
    gids = lax.broadcasted_iota(jnp.int32, (G, PKN), 0)
    sums = jnp.zeros((G, C), jnp.float32)
    cnt = jnp.zeros((G, 1), jnp.float32)
    for j in range(8):
        oh = (gids == batchj_ref[j:j + 1, :]).astype(jnp.float32)
        sj = jnp.dot(oh, h2n, preferred_element_type=jnp.float32)
        sums = sums + sj[:, C * j:C * (j + 1)]
        cnt = cnt + jnp.sum(oh, axis=1, keepdims=True)
    mean = sums / jnp.maximum(cnt, 1.0) + b_ref[...] * jnp.minimum(cnt, 1.0)
    m = jnp.max(mean, axis=1, keepdims=True)
    lse = jnp.log(jnp.sum(jnp.exp(mean - m), axis=1, keepdims=True)) + m
    out_ref[...] = mean - lse


def _final(p_pk, u_pk, dis_pk, batchj, b2):
    return pl.pallas_call(
        _final_body,
        out_shape=jax.ShapeDtypeStruct((G, C), jnp.float32),
    )(p_pk, u_pk, dis_pk, batchj, b2)


def kernel(x, edge_index, batch, W, b):
    ei3 = edge_index.reshape(2, CH_TOT, CHUNK)
    pad3 = jnp.full((2, PADCH, CHUNK), N, jnp.int32)
    x8 = x.reshape(PKN, 1024)
    batchj = batch.reshape(PKN, 8).T
    b2 = b.reshape(1, C)

    degp = _deg_kernel(ei3, pad3)
    u0_pk, dis_pk = _prep(x8, W, degp.reshape(2, PK, 128))
    pA = _round_kernel(u0_pk.reshape(NP, C), ei3, pad3)
    u1_pk = _mid(pA.reshape(2, PK, 128), u0_pk, dis_pk)
    pB = _round_kernel(u1_pk.reshape(NP, C), ei3, pad3)
    return _final(pB.reshape(2, PK, 128), u1_pk, dis_pk, batchj, b2)


# speedup vs baseline: 1.0939x; 1.0486x over previous
import jax
import jax.numpy as jnp
from jax import lax
from jax.experimental import pallas as pl
from jax.experimental.pallas import tpu as pltpu
from jax.experimental.pallas import tpu_sc as plsc

N = 10000
NP = 10016
E = 320000
C = 16
G = 128
NCORES = 2
NSUB = 16
NTILES = NCORES * NSUB
CHUNK = 128
CH_TOT = E // CHUNK
NCH0 = 88
NCH1 = 70
NCHMAX = max(NCH0, NCH1)
C1BASE = NSUB * NCH0
T31BASE = C1BASE + 15 * NCH1
BT31 = CH_TOT - T31BASE
PADCH = T31BASE + NCH1 - CH_TOT
GSZ = 16
RPT = NP // NSUB

_MESH = plsc.VectorSubcoreMesh(
    core_axis_name="c", subcore_axis_name="s",
    num_cores=NCORES, num_subcores=NSUB)


def _stage_indices(ei3, pad3, which, dst, cid, sid):
    @pl.when(cid == 0)
    def _():
        pltpu.sync_copy(ei3.at[which, pl.ds(sid * NCH0, NCH0)],
                        dst.at[pl.ds(0, NCH0)])

    @pl.when((cid == 1) & (sid < NSUB - 1))
    def _():
        pltpu.sync_copy(ei3.at[which, pl.ds(C1BASE + sid * NCH1, NCH1)],
                        dst.at[pl.ds(0, NCH1)])

    @pl.when((cid == 1) & (sid == NSUB - 1))
    def _():
        pltpu.sync_copy(ei3.at[which, pl.ds(T31BASE, BT31)],
                        dst.at[pl.ds(0, BT31)])
        pltpu.sync_copy(pad3.at[which], dst.at[pl.ds(BT31, PADCH)])


def _make_sc_body(with_gather):
    def body(*refs):
        if with_gather:
            (u_hbm, ei3, pad3, out_hbm,
             row_v, col_v, buf, zbuf, acc, u_sh, gsem, ssem) = refs
        else:
            ei3, pad3, out_hbm, col_v, buf, zbuf, acc, ssem = refs
        cid = lax.axis_index("c")
        sid = lax.axis_index("s")
        zeros16 = jnp.zeros((16,), jnp.float32)

        def zero_body(i, carry):
            zbuf[i, :] = zeros16
            return carry
        lax.fori_loop(0, RPT, zero_body, 0)
        pltpu.sync_copy(zbuf, acc.at[pl.ds(sid * RPT, RPT), :])
        if with_gather:
            pltpu.sync_copy(u_hbm.at[pl.ds(sid * RPT, RPT), :],
                            u_sh.at[pl.ds(sid * RPT, RPT), :])
            _stage_indices(ei3, pad3, 0, row_v, cid, sid)
        else:
            ones16 = jnp.ones((16,), jnp.float32)

            def ones_body(i, carry):
                buf[i, :] = ones16
                return carry
            lax.fori_loop(0, CHUNK, ones_body, 0)
        _stage_indices(ei3, pad3, 1, col_v, cid, sid)
        plsc.subcore_barrier()

        if with_gather:
            def issue_g(g, par, size):
                for b in range(size):
                    pltpu.async_copy(u_sh.at[row_v.at[g * GSZ + b]],
                                     buf.at[par, b], gsem)

            def issue_s(g, par, size):
                for b in range(size):
                    pltpu.async_copy(buf.at[par, b],
                                     acc.at[col_v.at[g * GSZ + b]],
                                     ssem, add=True)

            def drain(sem, k):
                for _ in range(k):
                    pltpu.make_async_copy(u_hbm.at[pl.ds(0, CHUNK), :],
                                          buf.at[0, 0], sem).wait()

            def pipeline(nch):
                ngf, tail = nch // GSZ, nch % GSZ
                tail_par = ngf % 2
                issue_g(0, 0, GSZ)

                def g_body(g, carry):
                    par = lax.rem(g, 2)
                    drain(gsem, GSZ)
                    issue_s(g, par, GSZ)

                    @pl.when(g >= 1)
                    def _():
                        drain(ssem, GSZ)

                    @pl.when(g + 1 < ngf)
                    def _():
                        issue_g(g + 1, 1 - par, GSZ)
                    return carry
                lax.fori_loop(0, ngf, g_body, 0)
                if tail:
                    issue_g(ngf, tail_par, tail)
                    drain(ssem, GSZ)
                    drain(gsem, tail)
                    issue_s(ngf, tail_par, tail)
                    drain(ssem, tail)
                else:
                    drain(ssem, GSZ)

            @pl.when(cid == 0)
            def _():
                pipeline(NCH0)

            @pl.when(cid == 1)
            def _():
                pipeline(NCH1)
        else:
            nch_t = jnp.where(cid == 0, NCH0, NCH1)

            def chunk_body(j, carry):
                pltpu.async_copy(buf, acc.at[col_v.at[j]], ssem, add=True)
                return carry
            lax.fori_loop(0, nch_t, chunk_body, 0)

            def drain_body(j, carry):
                pltpu.make_async_copy(
                    buf, acc.at[pl.ds(0, CHUNK), :], ssem).wait()
                return carry
            lax.fori_loop(0, nch_t, drain_body, 0)
        plsc.subcore_barrier()
        pltpu.sync_copy(acc.at[pl.ds(sid * RPT, RPT), :],
                        out_hbm.at[cid, pl.ds(sid * RPT, RPT), :])
    return body


_round_kernel = pl.kernel(
    _make_sc_body(True),
    out_type=jax.ShapeDtypeStruct((NCORES, NP, C), jnp.float32),
    mesh=_MESH,
    compiler_params=pltpu.CompilerParams(use_tc_tiling_on_sc=False),
    scratch_types=[
        pltpu.VMEM((NCHMAX, CHUNK), jnp.int32),
        pltpu.VMEM((NCHMAX, CHUNK), jnp.int32),
        pltpu.VMEM((2, GSZ, CHUNK, C), jnp.float32),
        pltpu.VMEM((RPT, C), jnp.float32),
        pltpu.VMEM_SHARED((NP, C), jnp.float32),
        pltpu.VMEM_SHARED((NP, C), jnp.float32),
        pltpu.SemaphoreType.DMA,
        pltpu.SemaphoreType.DMA,
    ],
)

_deg_kernel = pl.kernel(
    _make_sc_body(False),
    out_type=jax.ShapeDtypeStruct((NCORES, NP, C), jnp.float32),
    mesh=_MESH,
    compiler_params=pltpu.CompilerParams(use_tc_tiling_on_sc=False),
    scratch_types=[
        pltpu.VMEM((NCHMAX, CHUNK), jnp.int32),
        pltpu.VMEM((CHUNK, C), jnp.float32),
        pltpu.VMEM((RPT, C), jnp.float32),
        pltpu.VMEM_SHARED((NP, C), jnp.float32),
        pltpu.SemaphoreType.DMA,
    ],
)


PK = NP * C // 128
PKN = N * C // 128


def _prep_body(x_ref, w_ref, degp_ref, u0_ref, dis_ref):
    dis = lax.rsqrt(degp_ref[0] + degp_ref[1] + 1.0)
    w8 = jnp.concatenate([w_ref[...]] * 8, axis=1)
    yw = jnp.dot(x_ref[...], w8, preferred_element_type=jnp.float32)
    y3 = yw.reshape(PKN, 8, 128)
    blk = lax.broadcasted_iota(jnp.int32, (1, 8, 128), 2) // C
    sub = lax.broadcasted_iota(jnp.int32, (1, 8, 128), 1)
    mask = (blk == sub).astype(jnp.float32)
    ypk = jnp.sum(y3 * mask, axis=1)
    u0_ref[pl.ds(0, PKN), :] = dis[:PKN, :] * ypk
    u0_ref[pl.ds(PKN, PK - PKN), :] = jnp.zeros((PK - PKN, 128), jnp.float32)
    dis_ref[...] = dis


def _prep(x, W, degp_pk):
    return pl.pallas_call(
        _prep_body,
        out_shape=(jax.ShapeDtypeStruct((PK, 128), jnp.float32),
                   jax.ShapeDtypeStruct((PK, 128), jnp.float32)),
    )(x, W, degp_pk)


def _mid_body(p_ref, u_ref, dis_ref, out_ref):
    d = dis_ref[...]
    out_ref[...] = d * d * (p_ref[0] + p_ref[1] + u_ref[...])


def _mid(p_pk, u_pk, dis_pk):
    return pl.pallas_call(
        _mid_body,
        out_shape=jax.ShapeDtypeStruct((PK, 128), jnp.float32),
    )(p_pk, u_pk, dis_pk)


def _final_body(p_ref, u_ref, dis_ref, batchj_ref, b_ref, out_ref):
    d = dis_ref[...]
    h2 = d * (p_ref[0] + p_ref[1] + u_ref[...])
    h2n = h2[:PKN, :]
    gids = lax.broadcasted_iota(jnp.int32, (G, PKN), 0)
    sums = jnp.zeros((G, C), jnp.float32)
    cnt = jnp.zeros((G, 1), jnp.float32)
    for j in range(8):
        oh = (gids == batchj_ref[j:j + 1, :]).astype(jnp.float32)
        sj = jnp.dot(oh, h2n, preferred_element_type=jnp.float32)
        sums = sums + sj[:, C * j:C * (j + 1)]
        cnt = cnt + jnp.sum(oh, axis=1, keepdims=True)
    mean = sums / jnp.maximum(cnt, 1.0) + b_ref[...] * jnp.minimum(cnt, 1.0)
    m = jnp.max(mean, axis=1, keepdims=True)
    lse = jnp.log(jnp.sum(jnp.exp(mean - m), axis=1, keepdims=True)) + m
    out_ref[...] = mean - lse


def _final(p_pk, u_pk, dis_pk, batchj, b2):
    return pl.pallas_call(
        _final_body,
        out_shape=jax.ShapeDtypeStruct((G, C), jnp.float32),
    )(p_pk, u_pk, dis_pk, batchj, b2)


def kernel(x, edge_index, batch, W, b):
    ei3 = edge_index.reshape(2, CH_TOT, CHUNK)
    pad3 = jnp.full((2, PADCH, CHUNK), N, jnp.int32)
    batchj = batch.reshape(PKN, 8).T
    b2 = b.reshape(1, C)

    degp = _deg_kernel(ei3, pad3)
    u0_pk, dis_pk = _prep(x, W, degp.reshape(2, PK, 128))
    pA = _round_kernel(u0_pk.reshape(NP, C), ei3, pad3)
    u1_pk = _mid(pA.reshape(2, PK, 128), u0_pk, dis_pk)
    pB = _round_kernel(u1_pk.reshape(NP, C), ei3, pad3)
    return _final(pB.reshape(2, PK, 128), u1_pk, dis_pk, batchj, b2)
